# baseline (speedup 1.0000x reference)
import functools

import jax
import jax.numpy as jnp
from jax.experimental import pallas as pl
from jax.experimental.pallas import tpu as pltpu

_B = 16384
_F = 13
_CHUNKS = 4
_CB = _B // _CHUNKS
_EPS = 1e-5


def _fused_kernel(
    x_ref, wembT_ref, bemb_ref, w1T_ref, b1_ref, g1_ref, be1_ref,
    w2T_ref, b2_ref, g2_ref, be2_ref, w3T_ref, b3_ref,
    wwT_ref, bw_ref, wcT_ref, bc_ref,
    out_ref,
    dots_ref, nrm2_ref, w1d_ref, wwf_ref,
    s1_ref, ss1_ref, sc1_ref, sh1_ref,
    s2_ref, ss2_ref, sc2_ref, sh2_ref,
):
    p = pl.program_id(0)
    c = pl.program_id(1)
    xc = x_ref[...]

    @pl.when(p == 0)
    def _phase0():
        emb = jnp.dot(xc, wembT_ref[...]) + bemb_ref[...]
        d_c = jnp.sum(emb * emb[:, 0:1], axis=0, keepdims=True)
        n_c = jnp.sum(emb * emb, axis=0, keepdims=True)

        @pl.when(c == 0)
        def _():
            dots_ref[...] = d_c
            nrm2_ref[...] = n_c

        @pl.when(c > 0)
        def _():
            dots_ref[...] = dots_ref[...] + d_c
            nrm2_ref[...] = nrm2_ref[...] + n_c

        @pl.when(c == _CHUNKS - 1)
        def _select():
            nrm2 = nrm2_ref[...]
            nrm = jnp.sqrt(nrm2)
            q = nrm[0:1, 0:1]
            sim = dots_ref[...] / (nrm * q + 1e-12)
            dist = 1.0 - sim
            dcol = jnp.transpose(dist)
            drow = jnp.broadcast_to(dist, (_F, _F))
            dself = jnp.broadcast_to(dcol, (_F, _F))
            kk = jax.lax.broadcasted_iota(jnp.int32, (_F, _F), 1)
            jj = jax.lax.broadcasted_iota(jnp.int32, (_F, _F), 0)
            before = (drow < dself) | ((drow == dself) & (kk < jj))
            rank = jnp.sum(before.astype(jnp.float32), axis=1,
                           keepdims=True)
            j0 = jax.lax.broadcasted_iota(jnp.int32, (_F, 1), 0)
            wide_m = ((rank >= float(_F - 6)) | (j0 == 0)).astype(jnp.float32)
            deep_m = 1.0 - wide_m
            lt = (kk < jj).astype(jnp.float32)
            pos_w = jnp.dot(lt, wide_m)
            pos_d = jnp.dot(lt, deep_m)
            cols7 = jax.lax.broadcasted_iota(
                jnp.int32, (_F, 7), 1).astype(jnp.float32)
            cols6 = jax.lax.broadcasted_iota(
                jnp.int32, (_F, 6), 1).astype(jnp.float32)
            s_wide = jnp.where(pos_w == cols7, wide_m, 0.0)
            s_deep = jnp.where(pos_d == cols6, deep_m, 0.0)
            w1d_ref[...] = jnp.dot(s_deep, w1T_ref[...])
            wwf_ref[...] = jnp.dot(s_wide, wwT_ref[...])

    @pl.when(p == 1)
    def _phase1():
        h1 = jnp.dot(xc, w1d_ref[...]) + b1_ref[...]

        @pl.when(c == 0)
        def _():
            s1_ref[...] = jnp.sum(h1, axis=0, keepdims=True)
            ss1_ref[...] = jnp.sum(h1 * h1, axis=0, keepdims=True)

        @pl.when(c > 0)
        def _():
            s1_ref[...] = s1_ref[...] + jnp.sum(h1, axis=0, keepdims=True)
            ss1_ref[...] = ss1_ref[...] + jnp.sum(h1 * h1, axis=0,
                                                  keepdims=True)

        @pl.when(c == _CHUNKS - 1)
        def _():
            m = s1_ref[...] * (1.0 / _B)
            var = ss1_ref[...] * (1.0 / _B) - m * m
            sc = g1_ref[...] * jax.lax.rsqrt(var + _EPS)
            sc1_ref[...] = sc
            sh1_ref[...] = be1_ref[...] - m * sc

    @pl.when(p == 2)
    def _phase2():
        h1 = jnp.dot(xc, w1d_ref[...]) + b1_ref[...]
        a1 = jnp.maximum(h1 * sc1_ref[...] + sh1_ref[...], 0.0)
        h2 = jnp.dot(a1, w2T_ref[...]) + b2_ref[...]

        @pl.when(c == 0)
        def _():
            s2_ref[...] = jnp.sum(h2, axis=0, keepdims=True)
            ss2_ref[...] = jnp.sum(h2 * h2, axis=0, keepdims=True)

        @pl.when(c > 0)
        def _():
            s2_ref[...] = s2_ref[...] + jnp.sum(h2, axis=0, keepdims=True)
            ss2_ref[...] = ss2_ref[...] + jnp.sum(h2 * h2, axis=0,
                                                  keepdims=True)

        @pl.when(c == _CHUNKS - 1)
        def _():
            m = s2_ref[...] * (1.0 / _B)
            var = ss2_ref[...] * (1.0 / _B) - m * m
            sc = g2_ref[...] * jax.lax.rsqrt(var + _EPS)
            sc2_ref[...] = sc
            sh2_ref[...] = be2_ref[...] - m * sc

    @pl.when(p == 3)
    def _phase3():
        h1 = jnp.dot(xc, w1d_ref[...]) + b1_ref[...]
        a1 = jnp.maximum(h1 * sc1_ref[...] + sh1_ref[...], 0.0)
        h2 = jnp.dot(a1, w2T_ref[...]) + b2_ref[...]
        a2 = jnp.maximum(h2 * sc2_ref[...] + sh2_ref[...], 0.0)
        dnn = jnp.dot(a2, w3T_ref[...]) + b3_ref[...]
        wide = jnp.dot(xc, wwf_ref[...]) + bw_ref[...]
        glu = dnn * jax.nn.sigmoid(wide)
        logit = jnp.dot(glu, wcT_ref[...]) + bc_ref[...]
        out_ref[pl.ds(c * _CB, _CB), :] = jax.nn.sigmoid(logit)


def _whole(shape):
    return pl.BlockSpec(shape, lambda p, c: (0, 0))


@functools.partial(jax.jit)
def kernel(x, W_emb, b_emb, W1, b1, g1, be1, W2, b2, g2, be2, W3, b3,
           Ww, bw, Wc, bc):
    f32 = jnp.float32
    row = lambda v: v.reshape(1, -1).astype(f32)
    args = (
        x.astype(f32),
        W_emb.T.astype(f32), row(b_emb),
        W1.T.astype(f32), row(b1), row(g1), row(be1),
        W2.T.astype(f32), row(b2), row(g2), row(be2),
        W3.T.astype(f32), row(b3),
        Ww.T.astype(f32), row(bw),
        Wc.T.astype(f32), bc.reshape(1, 1).astype(f32),
    )
    in_specs = [
        pl.BlockSpec((_CB, _F), lambda p, c: (c, 0)),
        _whole((_F, _F)), _whole((1, _F)),
        _whole((6, 64)), _whole((1, 64)), _whole((1, 64)), _whole((1, 64)),
        _whole((64, 32)), _whole((1, 32)), _whole((1, 32)), _whole((1, 32)),
        _whole((32, 64)), _whole((1, 64)),
        _whole((7, 64)), _whole((1, 64)),
        _whole((64, 1)), _whole((1, 1)),
    ]
    scratch = [
        pltpu.VMEM((1, _F), f32), pltpu.VMEM((1, _F), f32),
        pltpu.VMEM((_F, 64), f32), pltpu.VMEM((_F, 64), f32),
        pltpu.VMEM((1, 64), f32), pltpu.VMEM((1, 64), f32),
        pltpu.VMEM((1, 64), f32), pltpu.VMEM((1, 64), f32),
        pltpu.VMEM((1, 32), f32), pltpu.VMEM((1, 32), f32),
        pltpu.VMEM((1, 32), f32), pltpu.VMEM((1, 32), f32),
    ]
    return pl.pallas_call(
        _fused_kernel,
        grid=(4, _CHUNKS),
        in_specs=in_specs,
        out_specs=pl.BlockSpec((_B, 1), lambda p, c: (0, 0)),
        out_shape=jax.ShapeDtypeStruct((_B, 1), f32),
        scratch_shapes=scratch,
    )(*args)


# speedup vs baseline: 1.1828x; 1.1828x over previous
import functools

import jax
import jax.numpy as jnp
from jax.experimental import pallas as pl
from jax.experimental.pallas import tpu as pltpu

_B = 16384
_F = 13
_EPS = 1e-5


def _fused_kernel(
    x_ref, wembT_ref, bemb_ref, w1T_ref, b1_ref, g1_ref, be1_ref,
    w2T_ref, b2_ref, g2_ref, be2_ref, w3T_ref, b3_ref,
    wwT_ref, bw_ref, wcT_ref, bc_ref,
    out_ref,
    w1d_ref, wwf_ref, sc1_ref, sh1_ref, sc2_ref, sh2_ref,
):
    p = pl.program_id(0)
    xc = x_ref[...]

    @pl.when(p == 0)
    def _phase0():
        g = jax.lax.dot_general(xc, xc, (((0,), (0,)), ((), ())))
        s = jnp.sum(xc, axis=0, keepdims=True)
        a = wembT_ref[...]
        b = bemb_ref[...]
        ata = jax.lax.dot_general(a, g, (((0,), (0,)), ((), ())))
        m0 = jnp.dot(ata, a)
        u = jnp.dot(s, a)
        ut = jnp.transpose(u)
        bt = jnp.transpose(b)
        m = m0 + ut * b + bt * u + float(_B) * (bt * b)
        eye = (jax.lax.broadcasted_iota(jnp.int32, (_F, _F), 0)
               == jax.lax.broadcasted_iota(jnp.int32, (_F, _F), 1))
        nrm2 = jnp.sum(jnp.where(eye, m, 0.0), axis=1,
                       keepdims=True)
        dots = m[:, 0:1]
        nrm = jnp.sqrt(nrm2)
        q = nrm[0:1, 0:1]
        dist = 1.0 - dots / (nrm * q + 1e-12)
        drow = jnp.transpose(dist)
        dself = jnp.broadcast_to(dist, (_F, _F))
        dk = jnp.broadcast_to(drow, (_F, _F))
        kk = jax.lax.broadcasted_iota(jnp.int32, (_F, _F), 1)
        jj = jax.lax.broadcasted_iota(jnp.int32, (_F, _F), 0)
        before = (dk < dself) | ((dk == dself) & (kk < jj))
        rank = jnp.sum(before.astype(jnp.float32), axis=1,
                       keepdims=True)
        j0 = jax.lax.broadcasted_iota(jnp.int32, (_F, 1), 0)
        wide_m = ((rank >= float(_F - 6)) | (j0 == 0)).astype(jnp.float32)
        deep_m = 1.0 - wide_m
        lt = (kk < jj).astype(jnp.float32)
        pos_w = jnp.dot(lt, wide_m)
        pos_d = jnp.dot(lt, deep_m)
        cols7 = jax.lax.broadcasted_iota(
            jnp.int32, (_F, 7), 1).astype(jnp.float32)
        cols6 = jax.lax.broadcasted_iota(
            jnp.int32, (_F, 6), 1).astype(jnp.float32)
        s_wide = jnp.where(pos_w == cols7, wide_m, 0.0)
        s_deep = jnp.where(pos_d == cols6, deep_m, 0.0)
        w1d = jnp.dot(s_deep, w1T_ref[...])
        w1d_ref[...] = w1d
        wwf_ref[...] = jnp.dot(s_wide, wwT_ref[...])
        mx = s * (1.0 / _B)
        mu1 = jnp.dot(mx, w1d) + b1_ref[...]
        mxt = jnp.transpose(mx)
        cov = g * (1.0 / _B) - mxt * mx
        t = jnp.dot(cov, w1d)
        var1 = jnp.sum(w1d * t, axis=0, keepdims=True)
        sc = g1_ref[...] * jax.lax.rsqrt(var1 + _EPS)
        sc1_ref[...] = sc
        sh1_ref[...] = be1_ref[...] - mu1 * sc

    @pl.when(p == 1)
    def _phase1():
        h1 = jnp.dot(xc, w1d_ref[...]) + b1_ref[...]
        a1 = jnp.maximum(h1 * sc1_ref[...] + sh1_ref[...], 0.0)
        h2 = jnp.dot(a1, w2T_ref[...]) + b2_ref[...]
        m = jnp.sum(h2, axis=0, keepdims=True) * (1.0 / _B)
        var = jnp.sum(h2 * h2, axis=0, keepdims=True) * (1.0 / _B) - m * m
        sc = g2_ref[...] * jax.lax.rsqrt(var + _EPS)
        sc2_ref[...] = sc
        sh2_ref[...] = be2_ref[...] - m * sc

    @pl.when(p == 2)
    def _phase2():
        h1 = jnp.dot(xc, w1d_ref[...]) + b1_ref[...]
        a1 = jnp.maximum(h1 * sc1_ref[...] + sh1_ref[...], 0.0)
        h2 = jnp.dot(a1, w2T_ref[...]) + b2_ref[...]
        a2 = jnp.maximum(h2 * sc2_ref[...] + sh2_ref[...], 0.0)
        dnn = jnp.dot(a2, w3T_ref[...]) + b3_ref[...]
        wide = jnp.dot(xc, wwf_ref[...]) + bw_ref[...]
        glu = dnn * jax.nn.sigmoid(wide)
        logit = jnp.dot(glu, wcT_ref[...]) + bc_ref[...]
        out_ref[...] = jax.nn.sigmoid(logit)


def _whole(shape):
    return pl.BlockSpec(shape, lambda p: tuple(0 for _ in shape))


@functools.partial(jax.jit)
def kernel(x, W_emb, b_emb, W1, b1, g1, be1, W2, b2, g2, be2, W3, b3,
           Ww, bw, Wc, bc):
    f32 = jnp.float32
    row = lambda v: v.reshape(1, -1).astype(f32)
    args = (
        x.astype(f32),
        W_emb.T.astype(f32), row(b_emb),
        W1.T.astype(f32), row(b1), row(g1), row(be1),
        W2.T.astype(f32), row(b2), row(g2), row(be2),
        W3.T.astype(f32), row(b3),
        Ww.T.astype(f32), row(bw),
        Wc.T.astype(f32), bc.reshape(1, 1).astype(f32),
    )
    in_specs = [
        _whole((_B, _F)),
        _whole((_F, _F)), _whole((1, _F)),
        _whole((6, 64)), _whole((1, 64)), _whole((1, 64)), _whole((1, 64)),
        _whole((64, 32)), _whole((1, 32)), _whole((1, 32)), _whole((1, 32)),
        _whole((32, 64)), _whole((1, 64)),
        _whole((7, 64)), _whole((1, 64)),
        _whole((64, 1)), _whole((1, 1)),
    ]
    scratch = [
        pltpu.VMEM((_F, 64), f32), pltpu.VMEM((_F, 64), f32),
        pltpu.VMEM((1, 64), f32), pltpu.VMEM((1, 64), f32),
        pltpu.VMEM((1, 32), f32), pltpu.VMEM((1, 32), f32),
    ]
    return pl.pallas_call(
        _fused_kernel,
        grid=(3,),
        in_specs=in_specs,
        out_specs=_whole((_B, 1)),
        out_shape=jax.ShapeDtypeStruct((_B, 1), f32),
        scratch_shapes=scratch,
    )(*args)


# speedup vs baseline: 1.2057x; 1.0193x over previous
import functools

import jax
import jax.numpy as jnp
from jax.experimental import pallas as pl
from jax.experimental.pallas import tpu as pltpu

_B = 16384
_F = 13
_EPS = 1e-5

_NN = (((1,), (0,)), ((), ()))
_NT = (((1,), (1,)), ((), ()))
_TN = (((0,), (0,)), ((), ()))


def _fused_kernel(
    x_ref, wemb_ref, bemb_ref, w1_ref, b1_ref, g1_ref, be1_ref,
    w2_ref, b2_ref, g2_ref, be2_ref, w3_ref, b3_ref,
    ww_ref, bw_ref, wc_ref, bc_ref,
    out_ref,
    w1d_ref, wwf_ref, sc1_ref, sh1_ref, sc2_ref, sh2_ref,
):
    p = pl.program_id(0)
    xc = x_ref[...]

    @pl.when(p == 0)
    def _phase0():
        g = jax.lax.dot_general(xc, xc, _TN)
        s = jnp.sum(xc, axis=0, keepdims=True)
        w = wemb_ref[...]
        b = bemb_ref[...]
        m0 = jax.lax.dot_general(jnp.dot(w, g), w, _NT)
        u = jax.lax.dot_general(s, w, _NT)
        ut = jnp.transpose(u)
        bt = jnp.transpose(b)
        m = m0 + ut * b + bt * u + float(_B) * (bt * b)
        eye = (jax.lax.broadcasted_iota(jnp.int32, (_F, _F), 0)
               == jax.lax.broadcasted_iota(jnp.int32, (_F, _F), 1))
        nrm2 = jnp.sum(jnp.where(eye, m, 0.0), axis=1,
                       keepdims=True)
        dots = m[:, 0:1]
        nrm = jnp.sqrt(nrm2)
        q = nrm[0:1, 0:1]
        dist = 1.0 - dots / (nrm * q + 1e-12)
        drow = jnp.transpose(dist)
        dself = jnp.broadcast_to(dist, (_F, _F))
        dk = jnp.broadcast_to(drow, (_F, _F))
        kk = jax.lax.broadcasted_iota(jnp.int32, (_F, _F), 1)
        jj = jax.lax.broadcasted_iota(jnp.int32, (_F, _F), 0)
        before = (dk < dself) | ((dk == dself) & (kk < jj))
        rank = jnp.sum(before.astype(jnp.float32), axis=1,
                       keepdims=True)
        j0 = jax.lax.broadcasted_iota(jnp.int32, (_F, 1), 0)
        wide_m = ((rank >= float(_F - 6)) | (j0 == 0)).astype(jnp.float32)
        deep_m = 1.0 - wide_m
        lt = (kk < jj).astype(jnp.float32)
        pos_w = jnp.dot(lt, wide_m)
        pos_d = jnp.dot(lt, deep_m)
        cols7 = jax.lax.broadcasted_iota(
            jnp.int32, (_F, 7), 1).astype(jnp.float32)
        cols6 = jax.lax.broadcasted_iota(
            jnp.int32, (_F, 6), 1).astype(jnp.float32)
        s_wide = jnp.where(pos_w == cols7, wide_m, 0.0)
        s_deep = jnp.where(pos_d == cols6, deep_m, 0.0)
        w1d = jax.lax.dot_general(s_deep, w1_ref[...], _NT)
        w1d_ref[...] = w1d
        wwf_ref[...] = jax.lax.dot_general(s_wide, ww_ref[...], _NT)
        mx = s * (1.0 / _B)
        mu1 = jnp.dot(mx, w1d) + b1_ref[...]
        mxt = jnp.transpose(mx)
        cov = g * (1.0 / _B) - mxt * mx
        t = jnp.dot(cov, w1d)
        var1 = jnp.sum(w1d * t, axis=0, keepdims=True)
        sc = g1_ref[...] * jax.lax.rsqrt(var1 + _EPS)
        sc1_ref[...] = sc
        sh1_ref[...] = be1_ref[...] - mu1 * sc

    @pl.when(p == 1)
    def _phase1():
        h1 = jnp.dot(xc, w1d_ref[...]) + b1_ref[...]
        a1 = jnp.maximum(h1 * sc1_ref[...] + sh1_ref[...], 0.0)
        h2 = jax.lax.dot_general(a1, w2_ref[...], _NT) + b2_ref[...]
        m = jnp.sum(h2, axis=0, keepdims=True) * (1.0 / _B)
        var = jnp.sum(h2 * h2, axis=0, keepdims=True) * (1.0 / _B) - m * m
        sc = g2_ref[...] * jax.lax.rsqrt(var + _EPS)
        sc2_ref[...] = sc
        sh2_ref[...] = be2_ref[...] - m * sc

    @pl.when(p == 2)
    def _phase2():
        h1 = jnp.dot(xc, w1d_ref[...]) + b1_ref[...]
        a1 = jnp.maximum(h1 * sc1_ref[...] + sh1_ref[...], 0.0)
        h2 = jax.lax.dot_general(a1, w2_ref[...], _NT) + b2_ref[...]
        a2 = jnp.maximum(h2 * sc2_ref[...] + sh2_ref[...], 0.0)
        dnn = jax.lax.dot_general(a2, w3_ref[...], _NT) + b3_ref[...]
        wide = jnp.dot(xc, wwf_ref[...]) + bw_ref[...]
        glu = dnn * jax.nn.sigmoid(wide)
        logit = jnp.sum(glu * wc_ref[...], axis=1,
                        keepdims=True) + bc_ref[0, 0]
        out_ref[...] = jax.nn.sigmoid(logit)


def _whole(shape):
    return pl.BlockSpec(shape, lambda p: tuple(0 for _ in shape))


@functools.partial(jax.jit)
def kernel(x, W_emb, b_emb, W1, b1, g1, be1, W2, b2, g2, be2, W3, b3,
           Ww, bw, Wc, bc):
    f32 = jnp.float32
    row = lambda v: v.reshape(1, -1).astype(f32)
    args = (
        x.astype(f32),
        W_emb.astype(f32), row(b_emb),
        W1.astype(f32), row(b1), row(g1), row(be1),
        W2.astype(f32), row(b2), row(g2), row(be2),
        W3.astype(f32), row(b3),
        Ww.astype(f32), row(bw),
        Wc.astype(f32), bc.reshape(1, 1).astype(f32),
    )
    in_specs = [
        _whole((_B, _F)),
        _whole((_F, _F)), _whole((1, _F)),
        _whole((64, 6)), _whole((1, 64)), _whole((1, 64)), _whole((1, 64)),
        _whole((32, 64)), _whole((1, 32)), _whole((1, 32)), _whole((1, 32)),
        _whole((64, 32)), _whole((1, 64)),
        _whole((64, 7)), _whole((1, 64)),
        _whole((1, 64)),
        pl.BlockSpec(memory_space=pltpu.SMEM),
    ]
    scratch = [
        pltpu.VMEM((_F, 64), f32), pltpu.VMEM((_F, 64), f32),
        pltpu.VMEM((1, 64), f32), pltpu.VMEM((1, 64), f32),
        pltpu.VMEM((1, 32), f32), pltpu.VMEM((1, 32), f32),
    ]
    return pl.pallas_call(
        _fused_kernel,
        grid=(3,),
        in_specs=in_specs,
        out_specs=_whole((_B, 1)),
        out_shape=jax.ShapeDtypeStruct((_B, 1), f32),
        scratch_shapes=scratch,
    )(*args)


# speedup vs baseline: 1.2981x; 1.0767x over previous
import functools

import jax
import jax.numpy as jnp
from jax.experimental import pallas as pl
from jax.experimental.pallas import tpu as pltpu

_B = 16384
_F = 13
_EPS = 1e-5

_NT = (((1,), (1,)), ((), ()))


def _fused_kernel(
    x_ref, wemb_ref, bemb_ref, w1_ref, b1_ref, g1_ref, be1_ref,
    w2_ref, b2_ref, g2_ref, be2_ref, w3_ref, b3_ref,
    ww_ref, bw_ref, wc_ref, bc_ref,
    out_ref,
):
    x = x_ref[...]

    emb = jax.lax.dot_general(x, wemb_ref[...], _NT) + bemb_ref[...]
    dots = jnp.sum(emb * emb[:, 0:1], axis=0, keepdims=True)
    nrm2 = jnp.sum(emb * emb, axis=0, keepdims=True)
    nrm = jnp.sqrt(nrm2)
    q = nrm[0:1, 0:1]
    dist = 1.0 - dots / (nrm * q + 1e-12)
    dcol = jnp.transpose(dist)
    dself = jnp.broadcast_to(dcol, (_F, _F))
    dk = jnp.broadcast_to(dist, (_F, _F))
    kk = jax.lax.broadcasted_iota(jnp.int32, (_F, _F), 1)
    jj = jax.lax.broadcasted_iota(jnp.int32, (_F, _F), 0)
    before = (dk < dself) | ((dk == dself) & (kk < jj))
    rank = jnp.sum(before.astype(jnp.float32), axis=1, keepdims=True)
    j0 = jax.lax.broadcasted_iota(jnp.int32, (_F, 1), 0)
    wide_m = ((rank >= float(_F - 6)) | (j0 == 0)).astype(jnp.float32)
    deep_m = 1.0 - wide_m
    lt = (kk < jj).astype(jnp.float32)
    pos_w = jnp.dot(lt, wide_m)
    pos_d = jnp.dot(lt, deep_m)
    cols7 = jax.lax.broadcasted_iota(jnp.int32, (_F, 7), 1).astype(
        jnp.float32)
    cols6 = jax.lax.broadcasted_iota(jnp.int32, (_F, 6), 1).astype(
        jnp.float32)
    s_wide = jnp.where(pos_w == cols7, wide_m, 0.0)
    s_deep = jnp.where(pos_d == cols6, deep_m, 0.0)
    w1d = jax.lax.dot_general(s_deep, w1_ref[...], _NT)
    wwf = jax.lax.dot_general(s_wide, ww_ref[...], _NT)

    h1 = jnp.dot(x, w1d) + b1_ref[...]
    m1 = jnp.sum(h1, axis=0, keepdims=True) * (1.0 / _B)
    v1 = jnp.sum(h1 * h1, axis=0, keepdims=True) * (1.0 / _B) - m1 * m1
    sc1 = g1_ref[...] * jax.lax.rsqrt(v1 + _EPS)
    sh1 = be1_ref[...] - m1 * sc1
    a1 = jnp.maximum(h1 * sc1 + sh1, 0.0)

    h2 = jax.lax.dot_general(a1, w2_ref[...], _NT) + b2_ref[...]
    m2 = jnp.sum(h2, axis=0, keepdims=True) * (1.0 / _B)
    v2 = jnp.sum(h2 * h2, axis=0, keepdims=True) * (1.0 / _B) - m2 * m2
    sc2 = g2_ref[...] * jax.lax.rsqrt(v2 + _EPS)
    sh2 = be2_ref[...] - m2 * sc2
    a2 = jnp.maximum(h2 * sc2 + sh2, 0.0)

    dnn = jax.lax.dot_general(a2, w3_ref[...], _NT) + b3_ref[...]

    wide = jnp.dot(x, wwf) + bw_ref[...]
    glu = dnn * jax.nn.sigmoid(wide)
    logit = jnp.sum(glu * wc_ref[...], axis=1, keepdims=True) + bc_ref[0, 0]
    out_ref[...] = jax.nn.sigmoid(logit)


@functools.partial(jax.jit)
def kernel(x, W_emb, b_emb, W1, b1, g1, be1, W2, b2, g2, be2, W3, b3,
           Ww, bw, Wc, bc):
    f32 = jnp.float32
    row = lambda v: v.reshape(1, -1).astype(f32)
    args = (
        x.astype(f32),
        W_emb.astype(f32), row(b_emb),
        W1.astype(f32), row(b1), row(g1), row(be1),
        W2.astype(f32), row(b2), row(g2), row(be2),
        W3.astype(f32), row(b3),
        Ww.astype(f32), row(bw),
        Wc.astype(f32), bc.reshape(1, 1).astype(f32),
    )
    in_specs = [
        pl.BlockSpec((_B, _F), lambda: (0, 0)),
        pl.BlockSpec((_F, _F), lambda: (0, 0)),
        pl.BlockSpec((1, _F), lambda: (0, 0)),
        pl.BlockSpec((64, 6), lambda: (0, 0)),
        pl.BlockSpec((1, 64), lambda: (0, 0)),
        pl.BlockSpec((1, 64), lambda: (0, 0)),
        pl.BlockSpec((1, 64), lambda: (0, 0)),
        pl.BlockSpec((32, 64), lambda: (0, 0)),
        pl.BlockSpec((1, 32), lambda: (0, 0)),
        pl.BlockSpec((1, 32), lambda: (0, 0)),
        pl.BlockSpec((1, 32), lambda: (0, 0)),
        pl.BlockSpec((64, 32), lambda: (0, 0)),
        pl.BlockSpec((1, 64), lambda: (0, 0)),
        pl.BlockSpec((64, 7), lambda: (0, 0)),
        pl.BlockSpec((1, 64), lambda: (0, 0)),
        pl.BlockSpec((1, 64), lambda: (0, 0)),
        pl.BlockSpec(memory_space=pltpu.SMEM),
    ]
    return pl.pallas_call(
        _fused_kernel,
        in_specs=in_specs,
        out_specs=pl.BlockSpec((_B, 1), lambda: (0, 0)),
        out_shape=jax.ShapeDtypeStruct((_B, 1), f32),
    )(*args)


# speedup vs baseline: 1.4925x; 1.1497x over previous
import functools

import jax
import jax.numpy as jnp
from jax.experimental import pallas as pl
from jax.experimental.pallas import tpu as pltpu

_B = 16384
_F = 13
_EPS = 1e-5

_NT = (((1,), (1,)), ((), ()))


def _fused_kernel(
    x_ref, wemb_ref, bemb_ref, w1_ref, b1_ref, g1_ref, be1_ref,
    w2_ref, b2_ref, g2_ref, be2_ref, w3_ref, b3_ref,
    ww_ref, bw_ref, wc_ref, bc_ref,
    out_ref,
):
    x = x_ref[...]

    emb = jax.lax.dot_general(x, wemb_ref[...], _NT) + bemb_ref[...]
    dots = jnp.sum(emb * emb[:, 0:1], axis=0, keepdims=True)
    nrm2 = jnp.sum(emb * emb, axis=0, keepdims=True)
    nrm = jnp.sqrt(nrm2)
    q = nrm[0:1, 0:1]
    dist = 1.0 - dots / (nrm * q + 1e-12)
    dcol = jnp.transpose(dist)
    dself = jnp.broadcast_to(dcol, (_F, _F))
    dk = jnp.broadcast_to(dist, (_F, _F))
    kk = jax.lax.broadcasted_iota(jnp.int32, (_F, _F), 1)
    jj = jax.lax.broadcasted_iota(jnp.int32, (_F, _F), 0)
    before = (dk < dself) | ((dk == dself) & (kk < jj))
    rank = jnp.sum(before.astype(jnp.float32), axis=1, keepdims=True)
    j0 = jax.lax.broadcasted_iota(jnp.int32, (_F, 1), 0)
    wide_m = ((rank >= float(_F - 6)) | (j0 == 0)).astype(jnp.float32)
    deep_m = 1.0 - wide_m
    lt = (kk < jj).astype(jnp.float32)
    pos_w = jnp.dot(lt, wide_m)
    pos_d = jnp.dot(lt, deep_m)
    cols7 = jax.lax.broadcasted_iota(jnp.int32, (_F, 7), 1).astype(
        jnp.float32)
    cols6 = jax.lax.broadcasted_iota(jnp.int32, (_F, 6), 1).astype(
        jnp.float32)
    s_wide = jnp.where(pos_w == cols7, wide_m, 0.0)
    s_deep = jnp.where(pos_d == cols6, deep_m, 0.0)
    w1d = jax.lax.dot_general(s_deep, w1_ref[...], _NT)
    wwf = jax.lax.dot_general(s_wide, ww_ref[...], _NT)

    h1 = jnp.dot(x, w1d) + b1_ref[...]
    m1 = jnp.sum(h1, axis=0, keepdims=True) * (1.0 / _B)
    v1 = jnp.sum(h1 * h1, axis=0, keepdims=True) * (1.0 / _B) - m1 * m1
    sc1 = g1_ref[...] * jax.lax.rsqrt(v1 + _EPS)
    sh1 = be1_ref[...] - m1 * sc1
    a1 = jnp.maximum(h1 * sc1 + sh1, 0.0)

    h2 = jax.lax.dot_general(a1, w2_ref[...], _NT) + b2_ref[...]
    m2 = jnp.sum(h2, axis=0, keepdims=True) * (1.0 / _B)
    v2 = jnp.sum(h2 * h2, axis=0, keepdims=True) * (1.0 / _B) - m2 * m2
    sc2 = g2_ref[...] * jax.lax.rsqrt(v2 + _EPS)
    sh2 = be2_ref[...] - m2 * sc2
    a2 = jnp.maximum(h2 * sc2 + sh2, 0.0)

    dnn = jax.lax.dot_general(a2, w3_ref[...], _NT) + b3_ref[...]

    wide = jnp.dot(x, wwf) + bw_ref[...]
    glu = dnn * jax.nn.sigmoid(wide)
    logit = jnp.sum(glu * wc_ref[...], axis=1, keepdims=True) + bc_ref[0, 0]
    out_ref[...] = jnp.transpose(jax.nn.sigmoid(logit))


@functools.partial(jax.jit)
def kernel(x, W_emb, b_emb, W1, b1, g1, be1, W2, b2, g2, be2, W3, b3,
           Ww, bw, Wc, bc):
    f32 = jnp.float32
    row = lambda v: v.reshape(1, -1).astype(f32)
    args = (
        x.astype(f32),
        W_emb.astype(f32), row(b_emb),
        W1.astype(f32), row(b1), row(g1), row(be1),
        W2.astype(f32), row(b2), row(g2), row(be2),
        W3.astype(f32), row(b3),
        Ww.astype(f32), row(bw),
        Wc.astype(f32), bc.reshape(1, 1).astype(f32),
    )
    in_specs = [
        pl.BlockSpec((_B, _F), lambda: (0, 0)),
        pl.BlockSpec((_F, _F), lambda: (0, 0)),
        pl.BlockSpec((1, _F), lambda: (0, 0)),
        pl.BlockSpec((64, 6), lambda: (0, 0)),
        pl.BlockSpec((1, 64), lambda: (0, 0)),
        pl.BlockSpec((1, 64), lambda: (0, 0)),
        pl.BlockSpec((1, 64), lambda: (0, 0)),
        pl.BlockSpec((32, 64), lambda: (0, 0)),
        pl.BlockSpec((1, 32), lambda: (0, 0)),
        pl.BlockSpec((1, 32), lambda: (0, 0)),
        pl.BlockSpec((1, 32), lambda: (0, 0)),
        pl.BlockSpec((64, 32), lambda: (0, 0)),
        pl.BlockSpec((1, 64), lambda: (0, 0)),
        pl.BlockSpec((64, 7), lambda: (0, 0)),
        pl.BlockSpec((1, 64), lambda: (0, 0)),
        pl.BlockSpec((1, 64), lambda: (0, 0)),
        pl.BlockSpec(memory_space=pltpu.SMEM),
    ]
    return pl.pallas_call(
        _fused_kernel,
        in_specs=in_specs,
        out_specs=pl.BlockSpec((1, _B), lambda: (0, 0)),
        out_shape=jax.ShapeDtypeStruct((1, _B), f32),
    )(*args).reshape(_B, 1)
